# baseline (speedup 1.0000x reference)
import math

import jax
import jax.numpy as jnp
from jax.experimental import pallas as pl

LR = 160
HR = 320
F1 = 2 * HR
H1 = 8
C1 = F1 // H1
F2 = HR
H2 = 4
C2 = F2 // H2

_DN11 = (((1,), (1,)), ((), ()))
_DN00 = (((0,), (0,)), ((), ()))
_DN10 = (((1,), (0,)), ((), ()))

_BF16 = jnp.bfloat16


def _mm(a, b, dn):
    return jax.lax.dot_general(a, b, dn, preferred_element_type=jnp.float32)


def _attention(q, k16, v16, n_heads, c, xt=None, we=None):
    outs = []
    for h in range(n_heads):
        sl = slice(h * c, (h + 1) * c)
        qh = q[:, sl]
        logits = _mm(qh.astype(_BF16), k16[:, sl], _DN11)
        if xt is not None:
            weh = we[:, sl]
            uh = jnp.sum(qh * weh, axis=1, keepdims=True)
            logits = logits + uh * xt
        m = jnp.max(logits, axis=1, keepdims=True)
        p = jnp.exp(logits - m)
        den = jnp.sum(p, axis=1, keepdims=True) + 1e-16
        aggh = _mm(p.astype(_BF16), v16[:, sl], _DN10)
        if xt is not None:
            aggh = aggh + jnp.sum(p * xt, axis=1, keepdims=True) * weh
        outs.append(aggh / den)
    return jnp.concatenate(outs, axis=1)


def _graphnorm(x, w, b, ms):
    mean = jnp.mean(x, axis=0, keepdims=True)
    out = x - mean * ms
    var = jnp.mean(out * out, axis=0, keepdims=True)
    return w * out / jnp.sqrt(var + 1e-5) + b


def _body(x_ref, wq1_ref, bq1_ref, wk1_ref, bk1_ref, wv1_ref, bv1_ref,
          we1_ref, wsk1_ref, bsk1_ref, gn1w_ref, gn1b_ref, gn1ms_ref,
          wq2_ref, bq2_ref, wk2_ref, bk2_ref, wv2_ref, bv2_ref,
          wsk2_ref, bsk2_ref, gn2w_ref, gn2b_ref, gn2ms_ref,
          hr_ref, lr_ref):
    inv1 = 1.0 / math.sqrt(C1)
    inv2 = 1.0 / math.sqrt(C2)
    x = x_ref[...]
    x16 = x.astype(_BF16)

    q1 = (_mm(x16, wq1_ref[...], _DN10) + bq1_ref[...]) * inv1
    k1 = _mm(x16, wk1_ref[...], _DN10) + bk1_ref[...]
    v1 = _mm(x16, wv1_ref[...], _DN10) + bv1_ref[...]
    xt = x.T
    h1 = _attention(q1, k1.astype(_BF16), v1.astype(_BF16), H1, C1,
                    xt=xt, we=we1_ref[...])
    h1 = h1 + _mm(x16, wsk1_ref[...], _DN10) + bsk1_ref[...]
    h1 = _graphnorm(h1, gn1w_ref[...], gn1b_ref[...], gn1ms_ref[...])
    lr_x = h1 / jnp.sqrt(jnp.sum(h1 * h1, axis=1, keepdims=True))
    lr16 = lr_x.astype(_BF16)
    lr_ref[...] = jnp.maximum(_mm(lr16, lr16, _DN11), 0.0)

    q2 = (_mm(lr16, wq2_ref[...], _DN00) + bq2_ref[...]) * inv2
    k2 = _mm(lr16, wk2_ref[...], _DN00) + bk2_ref[...]
    v2 = _mm(lr16, wv2_ref[...], _DN00) + bv2_ref[...]
    h2 = _attention(q2, k2.astype(_BF16), v2.astype(_BF16), H2, C2)
    h2 = h2 + _mm(lr16, wsk2_ref[...], _DN00) + bsk2_ref[...]
    g = _graphnorm(h2, gn2w_ref[...], gn2b_ref[...], gn2ms_ref[...])
    gg = (g / jnp.sqrt(jnp.sum(g * g, axis=0, keepdims=True))).astype(_BF16)
    hr_ref[...] = jnp.maximum(_mm(gg, gg, _DN00), 0.0)


def kernel(x, Wq1, bq1, Wk1, bk1, Wv1, bv1, We1, Wsk1, bsk1, gn1w, gn1b,
           gn1ms, Wq2, bq2, Wk2, bk2, Wv2, bv2, Wsk2, bsk2, gn2w, gn2b,
           gn2ms):
    row = lambda a: a.reshape(1, -1)
    bf = lambda a: a.astype(_BF16)
    return pl.pallas_call(
        _body,
        out_shape=(
            jax.ShapeDtypeStruct((HR, HR), jnp.float32),
            jax.ShapeDtypeStruct((LR, LR), jnp.float32),
        ),
    )(x, bf(Wq1), row(bq1), bf(Wk1), row(bk1), bf(Wv1), row(bv1), We1,
      bf(Wsk1), row(bsk1), row(gn1w), row(gn1b), row(gn1ms), bf(Wq2),
      row(bq2), bf(Wk2), row(bk2), bf(Wv2), row(bv2), bf(Wsk2), row(bsk2),
      row(gn2w), row(gn2b), row(gn2ms))


# speedup vs baseline: 1.3170x; 1.3170x over previous
import math

import jax
import jax.numpy as jnp
from jax.experimental import pallas as pl

LR = 160
HR = 320
F1 = 2 * HR
H1 = 8
C1 = F1 // H1
F2 = HR
H2 = 4
C2 = F2 // H2

_DN11 = (((1,), (1,)), ((), ()))
_DN00 = (((0,), (0,)), ((), ()))
_DN10 = (((1,), (0,)), ((), ()))


def _mm(a, b, dn):
    return jax.lax.dot_general(a, b, dn, preferred_element_type=jnp.float32)


def _attention(q, k, v, n_heads, c, xt=None, we=None):
    outs = []
    for h in range(n_heads):
        sl = slice(h * c, (h + 1) * c)
        qh, kh, vh = q[:, sl], k[:, sl], v[:, sl]
        logits = _mm(qh, kh, _DN11)
        if xt is not None:
            weh = we[:, sl]
            uh = jnp.sum(qh * weh, axis=1, keepdims=True)
            logits = logits + uh * xt
        m = jnp.max(logits, axis=1, keepdims=True)
        p = jnp.exp(logits - m)
        den = jnp.sum(p, axis=1, keepdims=True) + 1e-16
        aggh = _mm(p, vh, _DN10)
        if xt is not None:
            aggh = aggh + jnp.sum(p * xt, axis=1, keepdims=True) * weh
        outs.append(aggh / den)
    return jnp.concatenate(outs, axis=1)


def _graphnorm(x, w, b, ms):
    mean = jnp.mean(x, axis=0, keepdims=True)
    out = x - mean * ms
    var = jnp.mean(out * out, axis=0, keepdims=True)
    return w * out / jnp.sqrt(var + 1e-5) + b


def _body(x_ref, w1_ref, w2_ref, vec1_ref, vec2_ref, hr_ref, lr_ref):
    inv1 = 1.0 / math.sqrt(C1)
    inv2 = 1.0 / math.sqrt(C2)
    x = x_ref[...]
    bq1 = vec1_ref[0:1, :]
    bk1 = vec1_ref[1:2, :]
    bv1 = vec1_ref[2:3, :]
    bsk1 = vec1_ref[3:4, :]
    gn1w = vec1_ref[4:5, :]
    gn1b = vec1_ref[5:6, :]
    gn1ms = vec1_ref[6:7, :]
    we1 = vec1_ref[7:8, :]
    bq2 = vec2_ref[0:1, :]
    bk2 = vec2_ref[1:2, :]
    bv2 = vec2_ref[2:3, :]
    bsk2 = vec2_ref[3:4, :]
    gn2w = vec2_ref[4:5, :]
    gn2b = vec2_ref[5:6, :]
    gn2ms = vec2_ref[6:7, :]

    q1 = (_mm(x, w1_ref[0], _DN10) + bq1) * inv1
    k1 = _mm(x, w1_ref[1], _DN10) + bk1
    v1 = _mm(x, w1_ref[2], _DN10) + bv1
    xt = x.T
    h1 = _attention(q1, k1, v1, H1, C1, xt=xt, we=we1)
    h1 = h1 + _mm(x, w1_ref[3], _DN10) + bsk1
    h1 = _graphnorm(h1, gn1w, gn1b, gn1ms)
    lr_x = h1 / jnp.sqrt(jnp.sum(h1 * h1, axis=1, keepdims=True))
    lr_ref[...] = jnp.maximum(_mm(lr_x, lr_x, _DN11), 0.0)

    q2 = (_mm(lr_x, w2_ref[0], _DN00) + bq2) * inv2
    k2 = _mm(lr_x, w2_ref[1], _DN00) + bk2
    v2 = _mm(lr_x, w2_ref[2], _DN00) + bv2
    h2 = _attention(q2, k2, v2, H2, C2)
    h2 = h2 + _mm(lr_x, w2_ref[3], _DN00) + bsk2
    g = _graphnorm(h2, gn2w, gn2b, gn2ms)
    gg = g / jnp.sqrt(jnp.sum(g * g, axis=0, keepdims=True))
    hr_ref[...] = jnp.maximum(_mm(gg, gg, _DN00), 0.0)


def kernel(x, Wq1, bq1, Wk1, bk1, Wv1, bv1, We1, Wsk1, bsk1, gn1w, gn1b,
           gn1ms, Wq2, bq2, Wk2, bk2, Wv2, bv2, Wsk2, bsk2, gn2w, gn2b,
           gn2ms):
    w1 = jnp.stack([Wq1, Wk1, Wv1, Wsk1])
    w2 = jnp.stack([Wq2, Wk2, Wv2, Wsk2])
    vec1 = jnp.stack([bq1, bk1, bv1, bsk1, gn1w, gn1b, gn1ms,
                      We1.reshape(-1)])
    vec2 = jnp.stack([bq2, bk2, bv2, bsk2, gn2w, gn2b, gn2ms])
    return pl.pallas_call(
        _body,
        out_shape=(
            jax.ShapeDtypeStruct((HR, HR), jnp.float32),
            jax.ShapeDtypeStruct((LR, LR), jnp.float32),
        ),
    )(x, w1, w2, vec1, vec2)


# speedup vs baseline: 2.6584x; 2.0185x over previous
import math

import jax
import jax.numpy as jnp
from jax.experimental import pallas as pl

LR = 160
HR = 320
F1 = 2 * HR
H1 = 8
C1 = F1 // H1
F2 = HR
H2 = 4
C2 = F2 // H2

_DN11 = (((1,), (1,)), ((), ()))
_DN00 = (((0,), (0,)), ((), ()))


def _dot(a, b):
    return jnp.dot(a, b, preferred_element_type=jnp.float32)


def _attention(q, k, v, n_heads, c, xt=None, we=None):
    inv = 1.0 / math.sqrt(c)
    outs = []
    for h in range(n_heads):
        sl = slice(h * c, (h + 1) * c)
        qh, kh, vh = q[:, sl] * inv, k[:, sl], v[:, sl]
        logits = jax.lax.dot_general(qh, kh, _DN11,
                                     preferred_element_type=jnp.float32)
        if xt is not None:
            weh = we[:, sl]
            uh = jnp.sum(qh * weh, axis=1, keepdims=True)
            logits = logits + uh * xt
        m = jnp.max(logits, axis=1, keepdims=True)
        p = jnp.exp(logits - m)
        den = jnp.sum(p, axis=1, keepdims=True) + 1e-16
        aggh = _dot(p, vh)
        if xt is not None:
            aggh = aggh + jnp.sum(p * xt, axis=1, keepdims=True) * weh
        outs.append(aggh / den)
    return jnp.concatenate(outs, axis=1)


def _graphnorm(x, w, b, ms):
    mean = jnp.mean(x, axis=0, keepdims=True)
    out = x - mean * ms
    var = jnp.mean(out * out, axis=0, keepdims=True)
    return w * out / jnp.sqrt(var + 1e-5) + b


def _body(x_ref, wq1_ref, bq1_ref, wk1_ref, bk1_ref, wv1_ref, bv1_ref,
          we1_ref, wsk1_ref, bsk1_ref, gn1w_ref, gn1b_ref, gn1ms_ref,
          wq2_ref, bq2_ref, wk2_ref, bk2_ref, wv2_ref, bv2_ref,
          wsk2_ref, bsk2_ref, gn2w_ref, gn2b_ref, gn2ms_ref,
          hr_ref, lr_ref):
    x = x_ref[...]
    row = lambda r: r[...].reshape(1, -1)

    q1 = _dot(x, wq1_ref[...]) + row(bq1_ref)
    k1 = _dot(x, wk1_ref[...]) + row(bk1_ref)
    v1 = _dot(x, wv1_ref[...]) + row(bv1_ref)
    xt = x.T
    h1 = _attention(q1, k1, v1, H1, C1, xt=xt, we=we1_ref[...])
    h1 = h1 + _dot(x, wsk1_ref[...]) + row(bsk1_ref)
    h1 = _graphnorm(h1, row(gn1w_ref), row(gn1b_ref), row(gn1ms_ref))
    lr_x = h1 / jnp.sqrt(jnp.sum(h1 * h1, axis=1, keepdims=True))
    lr_ref[...] = jnp.maximum(
        jax.lax.dot_general(lr_x, lr_x, _DN11,
                            preferred_element_type=jnp.float32), 0.0)

    q2 = jax.lax.dot_general(lr_x, wq2_ref[...], _DN00,
                             preferred_element_type=jnp.float32) + row(bq2_ref)
    k2 = jax.lax.dot_general(lr_x, wk2_ref[...], _DN00,
                             preferred_element_type=jnp.float32) + row(bk2_ref)
    v2 = jax.lax.dot_general(lr_x, wv2_ref[...], _DN00,
                             preferred_element_type=jnp.float32) + row(bv2_ref)
    h2 = _attention(q2, k2, v2, H2, C2)
    h2 = h2 + jax.lax.dot_general(lr_x, wsk2_ref[...], _DN00,
                                  preferred_element_type=jnp.float32) \
            + row(bsk2_ref)
    g = _graphnorm(h2, row(gn2w_ref), row(gn2b_ref), row(gn2ms_ref))
    gg = g / jnp.sqrt(jnp.sum(g * g, axis=0, keepdims=True))
    hr_ref[...] = jnp.maximum(
        jax.lax.dot_general(gg, gg, _DN00,
                            preferred_element_type=jnp.float32), 0.0)


def kernel(x, Wq1, bq1, Wk1, bk1, Wv1, bv1, We1, Wsk1, bsk1, gn1w, gn1b,
           gn1ms, Wq2, bq2, Wk2, bk2, Wv2, bv2, Wsk2, bsk2, gn2w, gn2b,
           gn2ms):
    return pl.pallas_call(
        _body,
        out_shape=(
            jax.ShapeDtypeStruct((HR, HR), jnp.float32),
            jax.ShapeDtypeStruct((LR, LR), jnp.float32),
        ),
    )(x, Wq1, bq1, Wk1, bk1, Wv1, bv1, We1, Wsk1, bsk1,
      gn1w, gn1b, gn1ms, Wq2, bq2, Wk2, bk2, Wv2,
      bv2, Wsk2, bsk2, gn2w, gn2b, gn2ms)
